# baseline (speedup 1.0000x reference)
import jax
import jax.numpy as jnp
from jax.experimental import pallas as pl
from jax.experimental.pallas import tpu as pltpu

BATCH = 16
NUM_SEG = 64
SEG_LEN = 128
EMB = 128

N_BLK = 16


def _body(x_ref, seg_ref, pos_ref, out_ref):
    x = x_ref[...]
    seg = seg_ref[...]
    pos = pos_ref[...]
    out_ref[...] = x + seg[None, :, None, :] + pos[None, None, :, :]


def kernel(x, seg_table, pos_table):
    grid = (BATCH, NUM_SEG // N_BLK)
    return pl.pallas_call(
        _body,
        grid=grid,
        in_specs=[
            pl.BlockSpec((1, N_BLK, SEG_LEN, EMB), lambda b, n: (b, n, 0, 0)),
            pl.BlockSpec((N_BLK, EMB), lambda b, n: (n, 0)),
            pl.BlockSpec((SEG_LEN, EMB), lambda b, n: (0, 0)),
        ],
        out_specs=pl.BlockSpec((1, N_BLK, SEG_LEN, EMB), lambda b, n: (b, n, 0, 0)),
        out_shape=jax.ShapeDtypeStruct(x.shape, x.dtype),
    )(x, seg_table, pos_table)


# speedup vs baseline: 1.5287x; 1.5287x over previous
import jax
import jax.numpy as jnp
from jax.experimental import pallas as pl
from jax.experimental.pallas import tpu as pltpu

BATCH = 16
NUM_SEG = 64
SEG_LEN = 128
EMB = 128

N_BLK = 64


def _body(x_ref, seg_ref, pos_ref, out_ref):
    x = x_ref[...]
    seg = seg_ref[...]
    pos = pos_ref[...]
    out_ref[...] = x + seg[None, :, None, :] + pos[None, None, :, :]


def kernel(x, seg_table, pos_table):
    grid = (BATCH, NUM_SEG // N_BLK)
    return pl.pallas_call(
        _body,
        grid=grid,
        in_specs=[
            pl.BlockSpec((1, N_BLK, SEG_LEN, EMB), lambda b, n: (b, n, 0, 0)),
            pl.BlockSpec((N_BLK, EMB), lambda b, n: (n, 0)),
            pl.BlockSpec((SEG_LEN, EMB), lambda b, n: (0, 0)),
        ],
        out_specs=pl.BlockSpec((1, N_BLK, SEG_LEN, EMB), lambda b, n: (b, n, 0, 0)),
        out_shape=jax.ShapeDtypeStruct(x.shape, x.dtype),
        compiler_params=pltpu.CompilerParams(
            dimension_semantics=("arbitrary", "arbitrary"),
        ),
    )(x, seg_table, pos_table)
